# Initial kernel scaffold; baseline (speedup 1.0000x reference)
import jax
import jax.numpy as jnp
from jax.experimental import pallas as pl


def kernel(pts, rpn_w1, rpn_b1, rpn_w2, rpn_b2, rpn_wc, rpn_bc, rpn_wr, rpn_br, rcnn_w1, rcnn_b1, rcnn_wc, rcnn_bc, rcnn_wr, rcnn_br):
    raise NotImplementedError("write your pallas kernel here")


# speedup vs baseline: 1.3637x; 1.3637x over previous
from functools import partial

import jax
import jax.numpy as jnp
from jax.experimental import pallas as pl
from jax.experimental.pallas import tpu as pltpu

_SCORE_THRESH = 0.3
_C_IN = 3
_C_H1 = 32
_C_FEAT = 128
_C_REG = 7
_C_HEADS = 1 + _C_REG
_C_RCNN_IN = _C_FEAT + 2
_NUM_ROIS = 128


def _rpn_kernel(xyz_ref, w1_ref, b1_ref, w2_ref, b2_ref, wh_ref, bh_ref,
                feat_t_ref, cls_ref, reg_ref, segd_ref):
    xyz = xyz_ref[0]

    w1 = w1_ref[...]
    h1 = (xyz[:, 0:1] * w1[0:1, :]
          + xyz[:, 1:2] * w1[1:2, :]
          + xyz[:, 2:3] * w1[2:3, :]
          + b1_ref[...])
    h1 = jnp.maximum(h1, 0.0)

    feat = jnp.maximum(
        jnp.dot(h1, w2_ref[...], preferred_element_type=jnp.float32)
        + b2_ref[...], 0.0)

    heads = (jnp.dot(feat, wh_ref[...], preferred_element_type=jnp.float32)
             + bh_ref[...])

    cls = heads[:, 0:1]
    score = jax.nn.sigmoid(cls)
    mask = (score > _SCORE_THRESH).astype(jnp.float32)
    depth = jnp.sqrt(jnp.sum(xyz * xyz, axis=-1, keepdims=True))

    cls_ref[0] = cls
    reg_ref[0] = heads[:, 1:]
    segd_ref[0] = jnp.concatenate([mask, depth], axis=-1)
    feat_t_ref[0] = feat.T


def _rpn_forward(pts, w1, b1, w2, b2, wh, bh, tile):
    B, N, _ = pts.shape
    grid = (B, N // tile)

    def const(shape):
        return pl.BlockSpec(shape, lambda b, i: (0,) * len(shape))

    return pl.pallas_call(
        _rpn_kernel,
        grid=grid,
        in_specs=[pl.BlockSpec((1, tile, _C_IN), lambda b, i: (b, i, 0)),
                  const((_C_IN, _C_H1)), const((1, _C_H1)),
                  const((_C_H1, _C_FEAT)), const((1, _C_FEAT)),
                  const((_C_FEAT, _C_HEADS)), const((1, _C_HEADS))],
        out_specs=[pl.BlockSpec((1, _C_FEAT, tile), lambda b, i: (b, 0, i)),
                   pl.BlockSpec((1, tile, 1), lambda b, i: (b, i, 0)),
                   pl.BlockSpec((1, tile, _C_REG), lambda b, i: (b, i, 0)),
                   pl.BlockSpec((1, tile, 2), lambda b, i: (b, i, 0))],
        out_shape=[jax.ShapeDtypeStruct((B, _C_FEAT, N), jnp.float32),
                   jax.ShapeDtypeStruct((B, N, 1), jnp.float32),
                   jax.ShapeDtypeStruct((B, N, _C_REG), jnp.float32),
                   jax.ShapeDtypeStruct((B, N, 2), jnp.float32)],
        compiler_params=pltpu.CompilerParams(
            dimension_semantics=("parallel", "parallel"),
            vmem_limit_bytes=48 * 1024 * 1024),
    )(pts, w1, b1, w2, b2, wh, bh)


def _rcnn_kernel(x_ref, w1_ref, b1_ref, wh_ref, bh_ref, out_ref):
    h = jnp.maximum(
        jnp.dot(x_ref[...], w1_ref[...], preferred_element_type=jnp.float32)
        + b1_ref[...], 0.0)
    out_ref[...] = (jnp.dot(h, wh_ref[...], preferred_element_type=jnp.float32)
                    + bh_ref[...])


def _rcnn_forward(x_pool, w1, b1, wh, bh):
    R = x_pool.shape[0]
    half = R // 2

    def const(shape):
        return pl.BlockSpec(shape, lambda r: (0,) * len(shape))

    out = pl.pallas_call(
        _rcnn_kernel,
        grid=(2,),
        in_specs=[pl.BlockSpec((half, _C_RCNN_IN), lambda r: (r, 0)),
                  const((_C_RCNN_IN, 64)), const((1, 64)),
                  const((64, _C_HEADS)), const((1, _C_HEADS))],
        out_specs=pl.BlockSpec((half, _C_HEADS), lambda r: (r, 0)),
        out_shape=jax.ShapeDtypeStruct((R, _C_HEADS), jnp.float32),
        compiler_params=pltpu.CompilerParams(
            dimension_semantics=("parallel",)),
    )(x_pool, w1, b1, wh, bh)
    return out[:, 0:1], out[:, 1:]


def kernel(pts, rpn_w1, rpn_b1, rpn_w2, rpn_b2, rpn_wc, rpn_bc, rpn_wr,
           rpn_br, rcnn_w1, rcnn_b1, rcnn_wc, rcnn_bc, rcnn_wr, rcnn_br):
    B, N, _ = pts.shape
    tile = 2048
    while N % tile != 0:
        tile //= 2

    wh = jnp.concatenate([rpn_wc, rpn_wr], axis=1)
    bh = jnp.concatenate([rpn_bc, rpn_br], axis=1)

    feat_t, rpn_cls, rpn_reg, segd = _rpn_forward(
        pts, rpn_w1, rpn_b1, rpn_w2, rpn_b2, wh, bh, tile)

    rpn_scores_raw = rpn_cls[..., 0]
    seg_result = segd[..., 0]
    pts_depth = segd[..., 1]

    scores, idx = jax.lax.top_k(rpn_scores_raw, _NUM_ROIS)
    idx = idx.astype(jnp.int32)
    gather = lambda a: jnp.take_along_axis(a, idx[..., None], axis=1)
    centers = gather(pts) + gather(rpn_reg[..., :3])
    sizes = gather(rpn_reg[..., 3:])
    rois = jnp.concatenate([centers, sizes], axis=-1)

    feat_pool = jnp.take_along_axis(feat_t, idx[:, None, :], axis=2)
    feat_pool = feat_pool.transpose(0, 2, 1)
    seg_pool = jnp.take_along_axis(seg_result, idx, axis=1)[..., None]
    depth_pool = jnp.take_along_axis(pts_depth, idx, axis=1)[..., None]
    pooled = jnp.concatenate([feat_pool, seg_pool, depth_pool], axis=-1)

    rcnn_wh = jnp.concatenate([rcnn_wc, rcnn_wr], axis=1)
    rcnn_bh = jnp.concatenate([rcnn_bc, rcnn_br], axis=1)
    rcnn_cls, rcnn_reg = _rcnn_forward(
        pooled.reshape(B * _NUM_ROIS, _C_RCNN_IN),
        rcnn_w1, rcnn_b1, rcnn_wh, rcnn_bh)

    return {
        'backbone_xyz': pts,
        'backbone_features': feat_t,
        'rpn_cls': rpn_cls,
        'rpn_reg': rpn_reg,
        'rois': rois,
        'roi_scores_raw': scores,
        'seg_result': seg_result,
        'rcnn_cls': rcnn_cls,
        'rcnn_reg': rcnn_reg,
    }


# speedup vs baseline: 3.0589x; 2.2431x over previous
import functools

import jax
import jax.numpy as jnp
from jax.experimental import pallas as pl
from jax.experimental.pallas import tpu as pltpu

_SCORE_THRESH = 0.3
_C_IN = 3
_C_H1 = 32
_C_FEAT = 128
_C_REG = 7
_C_HEADS = 1 + _C_REG
_C_RCNN_IN = _C_FEAT + 2
_C_RCNN_H = 64
_NUM_ROIS = 128
_SUB = 8
_LANES = 128


def _rpn_kernel(xyz_ref, w1_ref, b1_ref, w2_ref, b2_ref, wh_ref, bh_ref,
                feat_t_ref, reg_ref, rows_ref):
    xyz = xyz_ref[0]

    w1 = w1_ref[...]
    h1 = (xyz[:, 0:1] * w1[0:1, :]
          + xyz[:, 1:2] * w1[1:2, :]
          + xyz[:, 2:3] * w1[2:3, :]
          + b1_ref[...])
    h1 = jnp.maximum(h1, 0.0)

    feat = jnp.maximum(
        jnp.dot(h1, w2_ref[...], preferred_element_type=jnp.float32)
        + b2_ref[...], 0.0)

    heads = (jnp.dot(feat, wh_ref[...], preferred_element_type=jnp.float32)
             + bh_ref[...])

    reg_ref[0] = heads[:, 1:]

    heads_t = heads.T
    cls_row = heads_t[0:1]
    score_row = jax.nn.sigmoid(cls_row)
    mask_row = (score_row > _SCORE_THRESH).astype(jnp.float32)
    rows_ref[0] = jnp.concatenate([cls_row, mask_row], axis=0)

    feat_t_ref[0] = feat.T


def _rpn_forward(pts, w1, b1, w2, b2, wh, bh, tile):
    B, N, _ = pts.shape
    grid = (B, N // tile)

    def const(shape):
        return pl.BlockSpec(shape, lambda b, i: (0,) * len(shape))

    return pl.pallas_call(
        _rpn_kernel,
        grid=grid,
        in_specs=[pl.BlockSpec((1, tile, _C_IN), lambda b, i: (b, i, 0)),
                  const((_C_IN, _C_H1)), const((1, _C_H1)),
                  const((_C_H1, _C_FEAT)), const((1, _C_FEAT)),
                  const((_C_FEAT, _C_HEADS)), const((1, _C_HEADS))],
        out_specs=[pl.BlockSpec((1, _C_FEAT, tile), lambda b, i: (b, 0, i)),
                   pl.BlockSpec((1, tile, _C_REG), lambda b, i: (b, i, 0)),
                   pl.BlockSpec((1, 2, tile), lambda b, i: (b, 0, i))],
        out_shape=[jax.ShapeDtypeStruct((B, _C_FEAT, N), jnp.float32),
                   jax.ShapeDtypeStruct((B, N, _C_REG), jnp.float32),
                   jax.ShapeDtypeStruct((B, 2, N), jnp.float32)],
        compiler_params=pltpu.CompilerParams(
            dimension_semantics=("parallel", "parallel"),
            vmem_limit_bytes=48 * 1024 * 1024),
    )(pts, w1, b1, w2, b2, wh, bh)


def _topk_kernel(s_ref, vals_ref, idx_ref, stack_ref, *, nv, k):
    G = s_ref.shape[0]
    stack_ref[...] = s_ref[...]
    s8 = jax.lax.broadcasted_iota(jnp.int32, (1, _SUB, _LANES), 1)
    lane = jax.lax.broadcasted_iota(jnp.int32, (1, _SUB, _LANES), 2)
    pos = (s8 * _LANES + lane).astype(jnp.float32)
    k_iota = jax.lax.broadcasted_iota(
        jnp.int32, (G, k), 1).astype(jnp.float32)
    big = float(nv * _SUB * _LANES * 2)

    def body(kk, carry):
        outv, outi, prev = carry
        colmax = jnp.full((G, _SUB, _LANES), -jnp.inf, jnp.float32)
        colarg = jnp.zeros((G, _SUB, _LANES), jnp.float32)
        for v in range(nv):
            sv = stack_ref[:, v * _SUB:(v + 1) * _SUB, :]
            fidx_v = float(v * _SUB * _LANES) + pos
            sv = jnp.where(fidx_v == prev, -jnp.inf, sv)
            stack_ref[:, v * _SUB:(v + 1) * _SUB, :] = sv
            upd = sv > colmax
            colmax = jnp.where(upd, sv, colmax)
            colarg = jnp.where(upd, float(v), colarg)
        m = jnp.max(colmax, axis=(1, 2), keepdims=True)
        fidx = colarg * float(_SUB * _LANES) + pos
        cand = jnp.where(colmax == m, fidx, big)
        imin = jnp.min(cand, axis=(1, 2), keepdims=True)
        outv = jnp.where(k_iota == kk, m[:, 0, :], outv)
        outi = jnp.where(k_iota == kk, imin[:, 0, :], outi)
        return outv, outi, imin

    outv = jnp.zeros((G, k), jnp.float32)
    outi = jnp.zeros((G, k), jnp.float32)
    prev = jnp.full((G, 1, 1), -1.0, jnp.float32)
    outv, outi, _ = jax.lax.fori_loop(0, k, body, (outv, outi, prev))
    vals_ref[...] = outv
    idx_ref[...] = outi.astype(jnp.int32)


def _topk(scores3, k):
    B = scores3.shape[0]
    G = 8
    while B % G != 0:
        G //= 2
    rows = scores3.shape[1]
    nv = rows // _SUB
    kern = functools.partial(_topk_kernel, nv=nv, k=k)
    vals, idx = pl.pallas_call(
        kern,
        grid=(B // G,),
        in_specs=[pl.BlockSpec((G, rows, _LANES), lambda b: (b, 0, 0))],
        out_specs=[pl.BlockSpec((G, k), lambda b: (b, 0)),
                   pl.BlockSpec((G, k), lambda b: (b, 0))],
        out_shape=[jax.ShapeDtypeStruct((B, k), jnp.float32),
                   jax.ShapeDtypeStruct((B, k), jnp.int32)],
        scratch_shapes=[pltpu.VMEM((G, rows, _LANES), jnp.float32)],
        compiler_params=pltpu.CompilerParams(
            dimension_semantics=("parallel",),
            vmem_limit_bytes=48 * 1024 * 1024),
    )(scores3)
    return vals, idx


def _tail_kernel(xyz_ref, w1_ref, b1_ref, w2_ref, b2_ref, wh_ref, bh_ref,
                 rw1_ref, rb1_ref, rwh_ref, rbh_ref,
                 rois_ref, cls_ref, reg_ref):
    xyz = xyz_ref[...]
    w1 = w1_ref[...]
    h1 = (xyz[:, 0:1] * w1[0:1, :]
          + xyz[:, 1:2] * w1[1:2, :]
          + xyz[:, 2:3] * w1[2:3, :]
          + b1_ref[...])
    h1 = jnp.maximum(h1, 0.0)
    feat = jnp.maximum(
        jnp.dot(h1, w2_ref[...], preferred_element_type=jnp.float32)
        + b2_ref[...], 0.0)
    heads = (jnp.dot(feat, wh_ref[...], preferred_element_type=jnp.float32)
             + bh_ref[...])

    score = jax.nn.sigmoid(heads[:, 0:1])
    mask = (score > _SCORE_THRESH).astype(jnp.float32)
    depth = jnp.sqrt(jnp.sum(xyz * xyz, axis=-1, keepdims=True))

    rois_ref[...] = jnp.concatenate(
        [xyz + heads[:, 1:4], heads[:, 4:8]], axis=-1)

    pooled = jnp.concatenate([feat, mask, depth], axis=-1)
    h = jnp.maximum(
        jnp.dot(pooled, rw1_ref[...], preferred_element_type=jnp.float32)
        + rb1_ref[...], 0.0)
    out = (jnp.dot(h, rwh_ref[...], preferred_element_type=jnp.float32)
           + rbh_ref[...])
    cls_ref[...] = out[:, 0:1]
    reg_ref[...] = out[:, 1:]


def _tail(xyz_sel, w1, b1, w2, b2, wh, bh, rw1, rb1, rwh, rbh):
    R = xyz_sel.shape[0]
    half = R // 2

    def const(shape):
        return pl.BlockSpec(shape, lambda r: (0,) * len(shape))

    return pl.pallas_call(
        _tail_kernel,
        grid=(2,),
        in_specs=[pl.BlockSpec((half, _C_IN), lambda r: (r, 0)),
                  const((_C_IN, _C_H1)), const((1, _C_H1)),
                  const((_C_H1, _C_FEAT)), const((1, _C_FEAT)),
                  const((_C_FEAT, _C_HEADS)), const((1, _C_HEADS)),
                  const((_C_RCNN_IN, _C_RCNN_H)), const((1, _C_RCNN_H)),
                  const((_C_RCNN_H, _C_HEADS)), const((1, _C_HEADS))],
        out_specs=[pl.BlockSpec((half, _C_REG), lambda r: (r, 0)),
                   pl.BlockSpec((half, 1), lambda r: (r, 0)),
                   pl.BlockSpec((half, _C_REG), lambda r: (r, 0))],
        out_shape=[jax.ShapeDtypeStruct((R, _C_REG), jnp.float32),
                   jax.ShapeDtypeStruct((R, 1), jnp.float32),
                   jax.ShapeDtypeStruct((R, _C_REG), jnp.float32)],
        compiler_params=pltpu.CompilerParams(
            dimension_semantics=("parallel",)),
    )(xyz_sel, w1, b1, w2, b2, wh, bh, rw1, rb1, rwh, rbh)


def kernel(pts, rpn_w1, rpn_b1, rpn_w2, rpn_b2, rpn_wc, rpn_bc, rpn_wr,
           rpn_br, rcnn_w1, rcnn_b1, rcnn_wc, rcnn_bc, rcnn_wr, rcnn_br):
    B, N, _ = pts.shape
    tile = 4096
    while N % tile != 0:
        tile //= 2

    wh = jnp.concatenate([rpn_wc, rpn_wr], axis=1)
    bh = jnp.concatenate([rpn_bc, rpn_br], axis=1)

    feat_t, rpn_reg, rows = _rpn_forward(
        pts, rpn_w1, rpn_b1, rpn_w2, rpn_b2, wh, bh, tile)

    rpn_cls = rows[:, 0:1, :].transpose(0, 2, 1)
    seg_result = rows[:, 1, :]
    scores3 = rows[:, 0, :].reshape(B, N // _LANES, _LANES)

    vals, idx = _topk(scores3, _NUM_ROIS)

    xyz_sel = jnp.take_along_axis(pts, idx[..., None], axis=1)

    rcnn_wh = jnp.concatenate([rcnn_wc, rcnn_wr], axis=1)
    rcnn_bh = jnp.concatenate([rcnn_bc, rcnn_br], axis=1)
    rois7, rcnn_cls, rcnn_reg = _tail(
        xyz_sel.reshape(B * _NUM_ROIS, _C_IN),
        rpn_w1, rpn_b1, rpn_w2, rpn_b2, wh, bh,
        rcnn_w1, rcnn_b1, rcnn_wh, rcnn_bh)

    return {
        'backbone_xyz': pts,
        'backbone_features': feat_t,
        'rpn_cls': rpn_cls,
        'rpn_reg': rpn_reg,
        'rois': rois7.reshape(B, _NUM_ROIS, _C_REG),
        'roi_scores_raw': vals,
        'seg_result': seg_result,
        'rcnn_cls': rcnn_cls,
        'rcnn_reg': rcnn_reg,
    }
